# Initial kernel scaffold; baseline (speedup 1.0000x reference)
import jax
import jax.numpy as jnp
from jax.experimental import pallas as pl


def kernel(x, batch_id, batch_size, W, b):
    raise NotImplementedError("write your pallas kernel here")


# speedup vs baseline: 3.5779x; 3.5779x over previous
import jax
import jax.numpy as jnp
from jax.experimental import pallas as pl

N, D, B = 100000, 128, 1024
RB = 1000


def _tc_body(ids_ref, x_ref, w_ref, b_ref, out_ref):
    i = pl.program_id(0)
    xb = x_ref[...]
    gate = jax.nn.sigmoid(
        jax.lax.dot_general(xb, w_ref[...], (((1,), (0,)), ((), ())),
                            preferred_element_type=jnp.float32)
        + b_ref[0]
    )
    weighted = xb * gate
    ids = ids_ref[0, 0, :]
    seg = jax.lax.broadcasted_iota(jnp.int32, (B, RB), 0)
    onehot_t = (seg == ids[None, :]).astype(jnp.float32)

    @pl.when(i == 0)
    def _():
        out_ref[...] = jnp.zeros_like(out_ref)

    out_ref[...] += jax.lax.dot_general(
        onehot_t, weighted, (((1,), (0,)), ((), ())),
        preferred_element_type=jnp.float32)


def kernel(x, batch_id, batch_size, W, b):
    ids3 = batch_id.reshape(N // RB, 1, RB)
    out = pl.pallas_call(
        _tc_body,
        grid=(N // RB,),
        in_specs=[
            pl.BlockSpec((1, 1, RB), lambda i: (i, 0, 0)),
            pl.BlockSpec((RB, D), lambda i: (i, 0)),
            pl.BlockSpec((D, 1), lambda i: (0, 0)),
            pl.BlockSpec((1,), lambda i: (0,)),
        ],
        out_specs=pl.BlockSpec((B, D), lambda i: (0, 0)),
        out_shape=jax.ShapeDtypeStruct((B, D), jnp.float32),
    )(ids3, x, W, b)
    return out + jnp.asarray(batch_size - B, dtype=out.dtype)
